# baseline (speedup 1.0000x reference)
import functools

import jax
import jax.numpy as jnp
from jax.experimental import pallas as pl

EMB_D = 64
K = 512
G = 8


def _vq_kernel(z_ref, e_ref, o_ref):
    e = e_ref[...]
    e2 = jnp.sum(e * e, axis=1, keepdims=True)
    for g in range(G):
        z = z_ref[g]
        scores = jax.lax.dot_general(
            e, z, (((1,), (0,)), ((), ())),
            preferred_element_type=jnp.float32,
        )
        z2 = jnp.sum(z * z, axis=0, keepdims=True)
        d = z2 + e2 - 2.0 * scores
        idx = jnp.argmin(d, axis=0)
        onehot = (
            jax.lax.broadcasted_iota(jnp.int32, d.shape, 0) == idx[None, :]
        ).astype(jnp.float32)
        o_ref[g] = jax.lax.dot_general(
            e, onehot, (((0,), (0,)), ((), ())),
            preferred_element_type=jnp.float32,
            precision=jax.lax.Precision.HIGHEST,
        )


@jax.jit
def kernel(z_e, e):
    B, C, H, W = z_e.shape
    HW = H * W
    z = z_e.reshape(B, C, HW)
    out = pl.pallas_call(
        _vq_kernel,
        grid=(B // G,),
        in_specs=[
            pl.BlockSpec((G, C, HW), lambda i: (i, 0, 0)),
            pl.BlockSpec((K, EMB_D), lambda i: (0, 0)),
        ],
        out_specs=pl.BlockSpec((G, C, HW), lambda i: (i, 0, 0)),
        out_shape=jax.ShapeDtypeStruct((B, C, HW), jnp.float32),
    )(z, e)
    return out.reshape(B, C, H, W)


# speedup vs baseline: 1.3018x; 1.3018x over previous
import functools

import jax
import jax.numpy as jnp
from jax.experimental import pallas as pl

EMB_D = 64
K = 512
G = 8


def _vq_kernel(z_ref, e_ref, o_ref):
    e = e_ref[...]
    e2 = jnp.sum(e * e, axis=1, keepdims=True)
    for g in range(G):
        z = z_ref[g]
        scores = jax.lax.dot_general(
            e, z, (((1,), (0,)), ((), ())),
            preferred_element_type=jnp.float32,
        )
        z2 = jnp.sum(z * z, axis=0, keepdims=True)
        d = z2 + e2 - 2.0 * scores
        m = jnp.min(d, axis=0, keepdims=True)
        onehot = (d == m).astype(jnp.float32)
        o_ref[g] = jax.lax.dot_general(
            e, onehot, (((0,), (0,)), ((), ())),
            preferred_element_type=jnp.float32,
        )


@jax.jit
def kernel(z_e, e):
    B, C, H, W = z_e.shape
    HW = H * W
    z = z_e.reshape(B, C, HW)
    out = pl.pallas_call(
        _vq_kernel,
        grid=(B // G,),
        in_specs=[
            pl.BlockSpec((G, C, HW), lambda i: (i, 0, 0)),
            pl.BlockSpec((K, EMB_D), lambda i: (0, 0)),
        ],
        out_specs=pl.BlockSpec((G, C, HW), lambda i: (i, 0, 0)),
        out_shape=jax.ShapeDtypeStruct((B, C, HW), jnp.float32),
    )(z, e)
    return out.reshape(B, C, H, W)


# speedup vs baseline: 1.3385x; 1.0282x over previous
import functools

import jax
import jax.numpy as jnp
from jax.experimental import pallas as pl

EMB_D = 64
K = 512
G = 8


def _vq_kernel(z_ref, e_ref, o_ref):
    e = e_ref[...]
    es = e * -2.0
    e2 = jnp.sum(e * e, axis=1, keepdims=True)
    for g in range(G):
        z = z_ref[g]
        d = e2 + jax.lax.dot_general(
            es, z, (((1,), (0,)), ((), ())),
            preferred_element_type=jnp.float32,
        )
        m = jnp.min(d, axis=0, keepdims=True)
        onehot = (d == m).astype(jnp.float32)
        o_ref[g] = jax.lax.dot_general(
            e, onehot, (((0,), (0,)), ((), ())),
            preferred_element_type=jnp.float32,
        )


@jax.jit
def kernel(z_e, e):
    B, C, H, W = z_e.shape
    HW = H * W
    z = z_e.reshape(B, C, HW)
    out = pl.pallas_call(
        _vq_kernel,
        grid=(B // G,),
        in_specs=[
            pl.BlockSpec((G, C, HW), lambda i: (i, 0, 0)),
            pl.BlockSpec((K, EMB_D), lambda i: (0, 0)),
        ],
        out_specs=pl.BlockSpec((G, C, HW), lambda i: (i, 0, 0)),
        out_shape=jax.ShapeDtypeStruct((B, C, HW), jnp.float32),
    )(z, e)
    return out.reshape(B, C, H, W)


# speedup vs baseline: 1.3637x; 1.0188x over previous
import functools

import jax
import jax.numpy as jnp
from jax.experimental import pallas as pl

EMB_D = 64
K = 512
G = 16


def _vq_kernel(z_ref, e_ref, o_ref):
    e = e_ref[...]
    es = e * -2.0
    e2 = jnp.sum(e * e, axis=1, keepdims=True)
    for g in range(G):
        z = z_ref[g]
        d = e2 + jax.lax.dot_general(
            es, z, (((1,), (0,)), ((), ())),
            preferred_element_type=jnp.float32,
        )
        m = jnp.min(d, axis=0, keepdims=True)
        onehot = (d == m).astype(jnp.float32)
        o_ref[g] = jax.lax.dot_general(
            e, onehot, (((0,), (0,)), ((), ())),
            preferred_element_type=jnp.float32,
        )


@jax.jit
def kernel(z_e, e):
    B, C, H, W = z_e.shape
    HW = H * W
    z = z_e.reshape(B, C, HW)
    out = pl.pallas_call(
        _vq_kernel,
        grid=(B // G,),
        in_specs=[
            pl.BlockSpec((G, C, HW), lambda i: (i, 0, 0)),
            pl.BlockSpec((K, EMB_D), lambda i: (0, 0)),
        ],
        out_specs=pl.BlockSpec((G, C, HW), lambda i: (i, 0, 0)),
        out_shape=jax.ShapeDtypeStruct((B, C, HW), jnp.float32),
    )(z, e)
    return out.reshape(B, C, H, W)


# speedup vs baseline: 1.3652x; 1.0011x over previous
import functools

import jax
import jax.numpy as jnp
from jax.experimental import pallas as pl
from jax.experimental.pallas import tpu as pltpu

EMB_D = 64
K = 512
G = 16


def _vq_kernel(z_ref, e_ref, o_ref):
    e = e_ref[...]
    es = e * -2.0
    e2 = jnp.sum(e * e, axis=1, keepdims=True)
    for g in range(G):
        z = z_ref[g]
        d = e2 + jax.lax.dot_general(
            es, z, (((1,), (0,)), ((), ())),
            preferred_element_type=jnp.float32,
        )
        m = jnp.min(d, axis=0, keepdims=True)
        onehot = (d == m).astype(jnp.float32)
        o_ref[g] = jax.lax.dot_general(
            e, onehot, (((0,), (0,)), ((), ())),
            preferred_element_type=jnp.float32,
        )


@jax.jit
def kernel(z_e, e):
    B, C, H, W = z_e.shape
    HW = H * W
    z = z_e.reshape(B, C, HW)
    out = pl.pallas_call(
        _vq_kernel,
        grid=(B // G,),
        in_specs=[
            pl.BlockSpec((G, C, HW), lambda i: (i, 0, 0)),
            pl.BlockSpec((K, EMB_D), lambda i: (0, 0)),
        ],
        out_specs=pl.BlockSpec((G, C, HW), lambda i: (i, 0, 0)),
        out_shape=jax.ShapeDtypeStruct((B, C, HW), jnp.float32),
        compiler_params=pltpu.CompilerParams(
            dimension_semantics=("parallel",),
        ),
    )(z, e)
    return out.reshape(B, C, H, W)
